# Initial kernel scaffold; baseline (speedup 1.0000x reference)
import jax
import jax.numpy as jnp
from jax.experimental import pallas as pl


def kernel(x, edge_attr, params, edge_index, batch, pat_idxs):
    raise NotImplementedError("write your pallas kernel here")


# speedup vs baseline: 1.1073x; 1.1073x over previous
import functools

import jax
import jax.numpy as jnp
from jax import lax
from jax.experimental import pallas as pl
from jax.experimental.pallas import tpu as pltpu
from jax.experimental.pallas import tpu_sc as plsc

_N = 10000
_D = 128
_E = 320000
_G = 8
_NPAD = 10240
_C = 128
_NC = 2
_NS = 16
_NW = _NC * _NS
_CPT = 79
_EPAD = _NW * _CPT * _C
_RPT = _NPAD // _NS

_mesh = plsc.VectorSubcoreMesh(
    core_axis_name="c", subcore_axis_name="s", num_cores=_NC, num_subcores=_NS)

_Z16 = lambda: jnp.zeros((16,), jnp.float32)


def _reduce_stage(s, c, local_v, stage_sh, red_v, tmp_v, out_hbm):
    pltpu.sync_copy(local_v, stage_sh.at[s])
    plsc.subcore_barrier()
    col = s * _RPT
    pltpu.sync_copy(stage_sh.at[0, pl.ds(col, _RPT)], red_v)

    def addrow(r, _):
        pltpu.sync_copy(stage_sh.at[r, pl.ds(col, _RPT)], tmp_v)

        def addv(j, _2):
            sl = pl.ds(j * 16, 16)
            red_v[sl] = red_v[sl] + tmp_v[sl]
            return 0

        lax.fori_loop(0, _RPT // 16, addv, 0)
        return 0

    lax.fori_loop(1, _NS, addrow, 0)
    pltpu.sync_copy(red_v, out_hbm.at[c, pl.ds(col, _RPT)])


@functools.partial(
    pl.kernel,
    out_type=(jax.ShapeDtypeStruct((_NC, _NPAD, _D), jnp.float32),
              jax.ShapeDtypeStruct((_NC, _NPAD), jnp.float32)),
    mesh=_mesh,
    compiler_params=pltpu.CompilerParams(needs_layout_passes=False),
    scratch_types=[
        pltpu.VMEM((_C,), jnp.int32),
        pltpu.VMEM((_C,), jnp.int32),
        pltpu.VMEM((_C, _D), jnp.float32),
        pltpu.VMEM((_NPAD,), jnp.float32),
        pltpu.VMEM((_RPT,), jnp.float32),
        pltpu.VMEM((_RPT,), jnp.float32),
        pltpu.VMEM_SHARED((_NPAD, _D), jnp.float32),
        pltpu.VMEM_SHARED((_NS, _NPAD), jnp.float32),
        pltpu.SemaphoreType.DMA,
    ],
)
def _mp_kernel(x_hbm, src_hbm, dst_hbm, agg_out, cnt_out,
               src_v, dst_v, rows_v, cnt_v, red_v, tmp_v, acc_sh, stage_sh,
               sem):
    c = lax.axis_index("c")
    s = lax.axis_index("s")
    wid = c * _NS + s
    ones16 = jnp.ones((16,), jnp.float32)

    def zrow(i, _):
        for j in range(_D // 16):
            rows_v[i, pl.ds(j * 16, 16)] = _Z16()
        return 0

    lax.fori_loop(0, _C, zrow, 0)
    for j in range(_RPT // _C):
        pltpu.sync_copy(rows_v, acc_sh.at[pl.ds(s * _RPT + j * _C, _C), :])

    def zcnt(i, _):
        cnt_v[pl.ds(i * 16, 16)] = _Z16()
        return 0

    lax.fori_loop(0, _NPAD // 16, zcnt, 0)
    plsc.subcore_barrier()

    def step(g, _):
        base = (wid * _CPT + g) * _C
        pltpu.sync_copy(src_hbm.at[pl.ds(base, _C)], src_v)
        pltpu.sync_copy(dst_hbm.at[pl.ds(base, _C)], dst_v)
        pltpu.async_copy(x_hbm.at[src_v], rows_v, sem).wait()
        pltpu.sync_copy(rows_v, acc_sh.at[dst_v], add=True)
        for j in range(_C // 16):
            idx = dst_v[pl.ds(j * 16, 16)]
            plsc.addupdate_scatter(cnt_v, [idx], ones16)
        return 0

    lax.fori_loop(0, _CPT, step, 0)
    plsc.subcore_barrier()

    _reduce_stage(s, c, cnt_v, stage_sh, red_v, tmp_v, cnt_out)

    for j in range(_RPT // _C):
        r0 = s * _RPT + j * _C
        pltpu.sync_copy(acc_sh.at[pl.ds(r0, _C), :],
                        agg_out.at[c, pl.ds(r0, _C), :])


_R = 1280


def _dense_body(agg_ref, cnt0_ref, cnt1_ref, x_ref, wl_ref, bl_ref, wr_ref,
                wc_ref, xn_ref, pr_ref):
    asum = agg_ref[0] + agg_ref[1]
    cd = jnp.maximum(cnt0_ref[0, 0] + cnt1_ref[0, 0], 1.0)
    mean = asum / cd[:, None]
    xn = (jnp.dot(mean, wl_ref[...], preferred_element_type=jnp.float32)
          + bl_ref[...]
          + jnp.dot(x_ref[...], wr_ref[...],
                    preferred_element_type=jnp.float32))
    xn = jnp.maximum(xn, 0.0)
    xn_ref[...] = xn
    pr_ref[...] = jnp.dot(xn, wc_ref[...], preferred_element_type=jnp.float32)


def _dense(agg, cnt, x, Wl, bl2, Wr, Wcat):
    cnt3 = cnt.reshape(_NC * 8, 1, _R)
    return pl.pallas_call(
        _dense_body,
        grid=(_NPAD // _R,),
        in_specs=[
            pl.BlockSpec((_NC, _R, _D), lambda i: (0, i, 0)),
            pl.BlockSpec((1, 1, _R), lambda i: (i, 0, 0)),
            pl.BlockSpec((1, 1, _R), lambda i: (i + 8, 0, 0)),
            pl.BlockSpec((_R, _D), lambda i: (i, 0)),
            pl.BlockSpec((_D, _D), lambda i: (0, 0)),
            pl.BlockSpec((1, _D), lambda i: (0, 0)),
            pl.BlockSpec((_D, _D), lambda i: (0, 0)),
            pl.BlockSpec((_D, 8), lambda i: (0, 0)),
        ],
        out_specs=[
            pl.BlockSpec((_R, _D), lambda i: (i, 0)),
            pl.BlockSpec((_R, 8), lambda i: (i, 0)),
        ],
        out_shape=[
            jax.ShapeDtypeStruct((_NPAD, _D), jnp.float32),
            jax.ShapeDtypeStruct((_NPAD, 8), jnp.float32),
        ],
    )(agg, cnt3, cnt3, x, Wl, bl2, Wr, Wcat)


def kernel(x, edge_attr, params, edge_index, batch, pat_idxs):
    f32 = jnp.float32
    x = x.at[:, :12].set(x[:, :12] / jnp.max(x[:, :12], axis=0, keepdims=True))
    b = batch
    src = edge_index[0]
    dst = edge_index[1]
    counts_total = jax.ops.segment_sum(jnp.ones_like(b), b, num_segments=_G)
    starts = jnp.concatenate([jnp.zeros((1,), counts_total.dtype),
                              jnp.cumsum(counts_total)[:-1]])
    pad = _EPAD - _E
    srcp = jnp.concatenate([src, jnp.zeros((pad,), jnp.int32)])
    dstp = jnp.concatenate([dst, jnp.zeros((pad,), jnp.int32)])
    evp = jnp.concatenate([jnp.ones((_E,), bool), jnp.zeros((pad,), bool)])
    node_valid = jnp.ones((_N,), bool)
    xpad = jnp.concatenate([x, jnp.zeros((_NPAD - _N, _D), f32)])
    xs = []
    for i in range(3):
        dstm = jnp.where(evp, dstp, _N)
        agg, cnt = _mp_kernel(xpad, srcp, dstm)
        bl2 = params['conv%d_bl' % i].reshape(1, _D)
        Wcat = jnp.concatenate(
            [params['pool%d_Wrel' % i], params['pool%d_Wroot' % i],
             jnp.zeros((_D, 6), f32)], axis=1)
        xn, pr = _dense(agg, cnt, xpad, params['conv%d_Wl' % i], bl2,
                        params['conv%d_Wr' % i], Wcat)
        sagg, _ = _mp_kernel(xn, srcp, dstm)
        s = ((sagg[0, :_N] + sagg[1, :_N]) @ params['pool%d_Wrel' % i][:, 0]
             + params['pool%d_brel' % i][0] + pr[:_N, 1])
        eff = jnp.where(node_valid, s, -jnp.inf)
        order = jnp.lexsort((-eff, b))
        g_of = b[order]
        pos = jnp.arange(_N, dtype=starts.dtype) - starts[g_of]
        vcount = jax.ops.segment_sum(node_valid.astype(f32), b,
                                     num_segments=_G)
        k = jnp.ceil(0.2 * vcount).astype(starts.dtype)
        sel_sorted = node_valid[order] & (pos < k[g_of])
        sel = jnp.zeros((_N,), bool).at[order].set(sel_sorted)
        xcore = jnp.where(sel[:, None], xn[:_N] * jnp.tanh(s)[:, None], 0.0)
        xpad = jnp.concatenate([xcore, jnp.zeros((_NPAD - _N, _D), f32)])
        evp = evp & sel[srcp] & sel[dstp]
        node_valid = sel
        bg = jnp.where(sel, b, _G)
        gmax = jax.ops.segment_max(xcore, bg, num_segments=_G + 1)[:_G]
        gsum = jax.ops.segment_sum(xcore, bg, num_segments=_G + 1)[:_G]
        gcnt = jax.ops.segment_sum(sel.astype(f32), b, num_segments=_G)
        gmean = gsum / jnp.maximum(gcnt, 1.0)[:, None]
        xs.append(jnp.concatenate([gmax, gmean], axis=1))
    h = xs[0] + xs[1] + xs[2]
    h = jax.nn.relu(h @ params['enc1_W'] + params['enc1_b'])
    h = jax.nn.relu(h @ params['enc2_W'] + params['enc2_b'])
    grade = jax.nn.log_softmax(h @ params['grade_W'] + params['grade_b'],
                               axis=1)
    hazard = jax.nn.sigmoid(h @ params['hazard_W'] + params['hazard_b']) * 6.0 - 3.0
    return (h, grade, hazard)


# speedup vs baseline: 1.3029x; 1.1766x over previous
import functools

import jax
import jax.numpy as jnp
from jax import lax
from jax.experimental import pallas as pl
from jax.experimental.pallas import tpu as pltpu
from jax.experimental.pallas import tpu_sc as plsc

_N = 10000
_D = 128
_E = 320000
_G = 8
_NPAD = 10240
_C = 128
_NC = 2
_NS = 16
_NW = _NC * _NS
_CPT = 79
_EPAD = _NW * _CPT * _C
_RPT = _NPAD // _NS

_mesh = plsc.VectorSubcoreMesh(
    core_axis_name="c", subcore_axis_name="s", num_cores=_NC, num_subcores=_NS)

_Z16 = lambda: jnp.zeros((16,), jnp.float32)


def _reduce_stage(s, c, local_v, stage_sh, red_v, tmp_v, out_hbm):
    pltpu.sync_copy(local_v, stage_sh.at[s])
    plsc.subcore_barrier()
    col = s * _RPT
    pltpu.sync_copy(stage_sh.at[0, pl.ds(col, _RPT)], red_v)

    def addrow(r, _):
        pltpu.sync_copy(stage_sh.at[r, pl.ds(col, _RPT)], tmp_v)

        def addv(j, _2):
            sl = pl.ds(j * 16, 16)
            red_v[sl] = red_v[sl] + tmp_v[sl]
            return 0

        lax.fori_loop(0, _RPT // 16, addv, 0)
        return 0

    lax.fori_loop(1, _NS, addrow, 0)
    pltpu.sync_copy(red_v, out_hbm.at[c, pl.ds(col, _RPT)])


@functools.partial(
    pl.kernel,
    out_type=(jax.ShapeDtypeStruct((_NC, _NPAD, _D), jnp.float32),
              jax.ShapeDtypeStruct((_NC, _NPAD), jnp.float32)),
    mesh=_mesh,
    compiler_params=pltpu.CompilerParams(needs_layout_passes=False),
    scratch_types=[
        pltpu.VMEM((_C,), jnp.int32),
        pltpu.VMEM((_C,), jnp.int32),
        pltpu.VMEM((_C, _D), jnp.float32),
        pltpu.VMEM((_NPAD,), jnp.float32),
        pltpu.VMEM((_RPT,), jnp.float32),
        pltpu.VMEM((_RPT,), jnp.float32),
        pltpu.VMEM_SHARED((_NPAD, _D), jnp.float32),
        pltpu.VMEM_SHARED((_NS, _NPAD), jnp.float32),
        pltpu.SemaphoreType.DMA,
    ],
)
def _mp_kernel(x_hbm, src_hbm, dst_hbm, agg_out, cnt_out,
               src_v, dst_v, rows_v, cnt_v, red_v, tmp_v, acc_sh, stage_sh,
               sem):
    c = lax.axis_index("c")
    s = lax.axis_index("s")
    wid = c * _NS + s
    ones16 = jnp.ones((16,), jnp.float32)

    def zrow(i, _):
        for j in range(_D // 16):
            rows_v[i, pl.ds(j * 16, 16)] = _Z16()
        return 0

    lax.fori_loop(0, _C, zrow, 0)
    for j in range(_RPT // _C):
        pltpu.sync_copy(rows_v, acc_sh.at[pl.ds(s * _RPT + j * _C, _C), :])

    def zcnt(i, _):
        cnt_v[pl.ds(i * 16, 16)] = _Z16()
        return 0

    lax.fori_loop(0, _NPAD // 16, zcnt, 0)
    plsc.subcore_barrier()

    def step(g, _):
        base = (wid * _CPT + g) * _C
        pltpu.sync_copy(src_hbm.at[pl.ds(base, _C)], src_v)
        pltpu.sync_copy(dst_hbm.at[pl.ds(base, _C)], dst_v)
        pltpu.async_copy(x_hbm.at[src_v], rows_v, sem).wait()
        pltpu.sync_copy(rows_v, acc_sh.at[dst_v], add=True)
        for j in range(_C // 16):
            idx = dst_v[pl.ds(j * 16, 16)]
            plsc.addupdate_scatter(cnt_v, [idx], ones16)
        return 0

    lax.fori_loop(0, _CPT, step, 0)
    plsc.subcore_barrier()

    _reduce_stage(s, c, cnt_v, stage_sh, red_v, tmp_v, cnt_out)

    for j in range(_RPT // _C):
        r0 = s * _RPT + j * _C
        pltpu.sync_copy(acc_sh.at[pl.ds(r0, _C), :],
                        agg_out.at[c, pl.ds(r0, _C), :])


_R = 1280


def _dense_body(agg_ref, cnt0_ref, cnt1_ref, x_ref, wl_ref, bl_ref, wr_ref,
                wc_ref, xn_ref, pr_ref):
    asum = agg_ref[0] + agg_ref[1]
    cd = jnp.maximum(cnt0_ref[0, 0] + cnt1_ref[0, 0], 1.0)
    mean = asum / cd[:, None]
    xn = (jnp.dot(mean, wl_ref[...], preferred_element_type=jnp.float32)
          + bl_ref[...]
          + jnp.dot(x_ref[...], wr_ref[...],
                    preferred_element_type=jnp.float32))
    xn = jnp.maximum(xn, 0.0)
    xn_ref[...] = xn
    pr_ref[...] = jnp.dot(xn, wc_ref[...], preferred_element_type=jnp.float32)


def _dense(agg, cnt, x, Wl, bl2, Wr, Wcat):
    cnt3 = cnt.reshape(_NC * 8, 1, _R)
    return pl.pallas_call(
        _dense_body,
        grid=(_NPAD // _R,),
        in_specs=[
            pl.BlockSpec((_NC, _R, _D), lambda i: (0, i, 0)),
            pl.BlockSpec((1, 1, _R), lambda i: (i, 0, 0)),
            pl.BlockSpec((1, 1, _R), lambda i: (i + 8, 0, 0)),
            pl.BlockSpec((_R, _D), lambda i: (i, 0)),
            pl.BlockSpec((_D, _D), lambda i: (0, 0)),
            pl.BlockSpec((1, _D), lambda i: (0, 0)),
            pl.BlockSpec((_D, _D), lambda i: (0, 0)),
            pl.BlockSpec((_D, 8), lambda i: (0, 0)),
        ],
        out_specs=[
            pl.BlockSpec((_R, _D), lambda i: (i, 0)),
            pl.BlockSpec((_R, 8), lambda i: (i, 0)),
        ],
        out_shape=[
            jax.ShapeDtypeStruct((_NPAD, _D), jnp.float32),
            jax.ShapeDtypeStruct((_NPAD, 8), jnp.float32),
        ],
    )(agg, cnt3, cnt3, x, Wl, bl2, Wr, Wcat)


def kernel(x, edge_attr, params, edge_index, batch, pat_idxs):
    f32 = jnp.float32
    x = x.at[:, :12].set(x[:, :12] / jnp.max(x[:, :12], axis=0, keepdims=True))
    b = batch
    src = edge_index[0]
    dst = edge_index[1]
    onehot = b[:, None] == jnp.arange(_G, dtype=b.dtype)[None, :]
    pad = _EPAD - _E
    srcp = jnp.concatenate([src, jnp.zeros((pad,), jnp.int32)])
    dstp = jnp.concatenate([dst, jnp.zeros((pad,), jnp.int32)])
    evp = jnp.concatenate([jnp.ones((_E,), bool), jnp.zeros((pad,), bool)])
    node_valid = jnp.ones((_N,), bool)
    xpad = jnp.concatenate([x, jnp.zeros((_NPAD - _N, _D), f32)])
    xs = []
    for i in range(3):
        dstm = jnp.where(evp, dstp, _N)
        agg, cnt = _mp_kernel(xpad, srcp, dstm)
        bl2 = params['conv%d_bl' % i].reshape(1, _D)
        Wcat = jnp.concatenate(
            [params['pool%d_Wrel' % i], params['pool%d_Wroot' % i],
             jnp.zeros((_D, 6), f32)], axis=1)
        xn, pr = _dense(agg, cnt, xpad, params['conv%d_Wl' % i], bl2,
                        params['conv%d_Wr' % i], Wcat)
        sagg, _ = _mp_kernel(xn, srcp, dstm)
        s = ((sagg[0, :_N] + sagg[1, :_N]) @ params['pool%d_Wrel' % i][:, 0]
             + params['pool%d_brel' % i][0] + pr[:_N, 1])
        bits = jax.lax.bitcast_convert_type(s, jnp.uint32)
        msk = jnp.where((bits >> 31).astype(jnp.int32) == 1,
                        jnp.uint32(0xFFFFFFFF), jnp.uint32(0x80000000))
        key = jnp.where(node_valid, bits ^ msk, jnp.uint32(0))
        vcount = jnp.sum(onehot & node_valid[:, None], axis=0).astype(f32)
        k = jnp.ceil(0.2 * vcount).astype(jnp.int32)

        def bs_body(_, lohi):
            lo, hi = lohi
            mid = lo + (hi - lo) // 2
            tn = jnp.sum(jnp.where(onehot, mid[None, :], jnp.uint32(0)),
                         axis=1)
            cnt = jnp.sum((onehot & (key > tn)[:, None]).astype(jnp.int32),
                          axis=0)
            ge = cnt >= k
            return jnp.where(ge, mid + 1, lo), jnp.where(ge, hi, mid)

        lo, hi = lax.fori_loop(0, 32, bs_body,
                               (jnp.zeros((_G,), jnp.uint32),
                                jnp.full((_G,), 0xFFFFFFFF, jnp.uint32)))
        Tn = jnp.sum(jnp.where(onehot, lo[None, :], jnp.uint32(0)), axis=1)
        selhard = node_valid & (key > Tn)
        tie = node_valid & (key == Tn)
        c1 = jnp.sum((onehot & selhard[:, None]).astype(jnp.int32), axis=0)
        m = k - c1
        tiecnt = jnp.sum((onehot & tie[:, None]).astype(jnp.int32), axis=0)
        base = jnp.concatenate([jnp.zeros((1,), jnp.int32),
                                jnp.cumsum(tiecnt)[:-1]])
        cs = jnp.cumsum(tie.astype(jnp.int32))
        basen = jnp.sum(jnp.where(onehot, base[None, :], 0), axis=1)
        mn = jnp.sum(jnp.where(onehot, m[None, :], 0), axis=1)
        sel = selhard | (tie & (cs - 1 - basen < mn))

        xcore = jnp.where(sel[:, None], xn[:_N] * jnp.tanh(s)[:, None], 0.0)
        xpad = jnp.concatenate([xcore, jnp.zeros((_NPAD - _N, _D), f32)])
        evp = evp & sel[srcp] & sel[dstp]
        node_valid = sel
        oh_sel = onehot & sel[:, None]
        gmax = jnp.stack([
            jnp.max(jnp.where(oh_sel[:, g][:, None], xcore, -jnp.inf), axis=0)
            for g in range(_G)])
        gsum = jnp.dot(oh_sel.astype(f32).T, xcore,
                       preferred_element_type=f32)
        gcnt = jnp.sum(oh_sel, axis=0).astype(f32)
        gmean = gsum / jnp.maximum(gcnt, 1.0)[:, None]
        xs.append(jnp.concatenate([gmax, gmean], axis=1))
    h = xs[0] + xs[1] + xs[2]
    h = jax.nn.relu(h @ params['enc1_W'] + params['enc1_b'])
    h = jax.nn.relu(h @ params['enc2_W'] + params['enc2_b'])
    grade = jax.nn.log_softmax(h @ params['grade_W'] + params['grade_b'],
                               axis=1)
    hazard = jax.nn.sigmoid(h @ params['hazard_W'] + params['hazard_b']) * 6.0 - 3.0
    return (h, grade, hazard)


# speedup vs baseline: 6.8619x; 5.2665x over previous
import functools

import jax
import jax.numpy as jnp
from jax import lax
from jax.experimental import pallas as pl
from jax.experimental.pallas import tpu as pltpu
from jax.experimental.pallas import tpu_sc as plsc

_N = 10000
_D = 128
_E = 320000
_G = 8
_NPAD = 10240
_C = 128
_NC = 2
_NS = 16
_NW = _NC * _NS
_CPT = 79
_EPAD = _NW * _CPT * _C
_RPT = _NPAD // _NS

_mesh = plsc.VectorSubcoreMesh(
    core_axis_name="c", subcore_axis_name="s", num_cores=_NC, num_subcores=_NS)

_Z16 = lambda: jnp.zeros((16,), jnp.float32)


def _reduce_stage(s, c, local_v, stage_sh, red_v, tmp_v, out_hbm):
    pltpu.sync_copy(local_v, stage_sh.at[s])
    plsc.subcore_barrier()
    col = s * _RPT
    pltpu.sync_copy(stage_sh.at[0, pl.ds(col, _RPT)], red_v)

    def addrow(r, _):
        pltpu.sync_copy(stage_sh.at[r, pl.ds(col, _RPT)], tmp_v)

        def addv(j, _2):
            sl = pl.ds(j * 16, 16)
            red_v[sl] = red_v[sl] + tmp_v[sl]
            return 0

        lax.fori_loop(0, _RPT // 16, addv, 0)
        return 0

    lax.fori_loop(1, _NS, addrow, 0)
    pltpu.sync_copy(red_v, out_hbm.at[c, pl.ds(col, _RPT)])


@functools.partial(
    pl.kernel,
    out_type=(jax.ShapeDtypeStruct((_NC, _NPAD, _D), jnp.float32),
              jax.ShapeDtypeStruct((_NC, _NPAD), jnp.float32)),
    mesh=_mesh,
    compiler_params=pltpu.CompilerParams(needs_layout_passes=False),
    scratch_types=[
        pltpu.VMEM((_C,), jnp.int32),
        pltpu.VMEM((_C,), jnp.int32),
        pltpu.VMEM((_C,), jnp.int32),
        pltpu.VMEM((_C, _D), jnp.float32),
        pltpu.VMEM((_NPAD,), jnp.float32),
        pltpu.VMEM((_NPAD,), jnp.float32),
        pltpu.VMEM((_RPT,), jnp.float32),
        pltpu.VMEM((_RPT,), jnp.float32),
        pltpu.VMEM_SHARED((_NPAD, _D), jnp.float32),
        pltpu.VMEM_SHARED((_NS, _NPAD), jnp.float32),
        pltpu.SemaphoreType.DMA,
    ],
)
def _mp_kernel(x_hbm, src_hbm, dst_hbm, sel_hbm, agg_out, cnt_out,
               src_v, dst_v, dm_v, rows_v, sv, cnt_v, red_v, tmp_v, acc_sh,
               stage_sh, sem):
    c = lax.axis_index("c")
    s = lax.axis_index("s")
    wid = c * _NS + s
    ones16 = jnp.ones((16,), jnp.float32)

    pltpu.sync_copy(sel_hbm, sv)

    def zrow(i, _):
        for j in range(_D // 16):
            rows_v[i, pl.ds(j * 16, 16)] = _Z16()
        return 0

    lax.fori_loop(0, _C, zrow, 0)
    for j in range(_RPT // _C):
        pltpu.sync_copy(rows_v, acc_sh.at[pl.ds(s * _RPT + j * _C, _C), :])

    def zcnt(i, _):
        cnt_v[pl.ds(i * 16, 16)] = _Z16()
        return 0

    lax.fori_loop(0, _NPAD // 16, zcnt, 0)
    plsc.subcore_barrier()

    def step(g, _):
        base = (wid * _CPT + g) * _C
        pltpu.sync_copy(src_hbm.at[pl.ds(base, _C)], src_v)
        pltpu.sync_copy(dst_hbm.at[pl.ds(base, _C)], dst_v)
        for j in range(_C // 16):
            sl = pl.ds(j * 16, 16)
            s16 = src_v[sl]
            d16 = dst_v[sl]
            ok = (plsc.load_gather(sv, [s16])
                  * plsc.load_gather(sv, [d16])) > 0.5
            dm16 = jnp.where(ok, d16, _N)
            dm_v[sl] = dm16
            plsc.addupdate_scatter(cnt_v, [dm16], ones16)
        pltpu.async_copy(x_hbm.at[src_v], rows_v, sem).wait()
        pltpu.sync_copy(rows_v, acc_sh.at[dm_v], add=True)
        return 0

    lax.fori_loop(0, _CPT, step, 0)
    plsc.subcore_barrier()

    _reduce_stage(s, c, cnt_v, stage_sh, red_v, tmp_v, cnt_out)

    for j in range(_RPT // _C):
        r0 = s * _RPT + j * _C
        pltpu.sync_copy(acc_sh.at[pl.ds(r0, _C), :],
                        agg_out.at[c, pl.ds(r0, _C), :])


_R = 1280


def _dense_body(agg_ref, cnt0_ref, cnt1_ref, x_ref, wl_ref, bl_ref, wr_ref,
                wc_ref, xn_ref, pr_ref):
    asum = agg_ref[0] + agg_ref[1]
    cd = jnp.maximum(cnt0_ref[0, 0] + cnt1_ref[0, 0], 1.0)
    mean = asum / cd[:, None]
    xn = (jnp.dot(mean, wl_ref[...], preferred_element_type=jnp.float32)
          + bl_ref[...]
          + jnp.dot(x_ref[...], wr_ref[...],
                    preferred_element_type=jnp.float32))
    xn = jnp.maximum(xn, 0.0)
    xn_ref[...] = xn
    pr_ref[...] = jnp.dot(xn, wc_ref[...], preferred_element_type=jnp.float32)


def _dense(agg, cnt, x, Wl, bl2, Wr, Wcat):
    cnt3 = cnt.reshape(_NC * 8, 1, _R)
    return pl.pallas_call(
        _dense_body,
        grid=(_NPAD // _R,),
        in_specs=[
            pl.BlockSpec((_NC, _R, _D), lambda i: (0, i, 0)),
            pl.BlockSpec((1, 1, _R), lambda i: (i, 0, 0)),
            pl.BlockSpec((1, 1, _R), lambda i: (i + 8, 0, 0)),
            pl.BlockSpec((_R, _D), lambda i: (i, 0)),
            pl.BlockSpec((_D, _D), lambda i: (0, 0)),
            pl.BlockSpec((1, _D), lambda i: (0, 0)),
            pl.BlockSpec((_D, _D), lambda i: (0, 0)),
            pl.BlockSpec((_D, 8), lambda i: (0, 0)),
        ],
        out_specs=[
            pl.BlockSpec((_R, _D), lambda i: (i, 0)),
            pl.BlockSpec((_R, 8), lambda i: (i, 0)),
        ],
        out_shape=[
            jax.ShapeDtypeStruct((_NPAD, _D), jnp.float32),
            jax.ShapeDtypeStruct((_NPAD, 8), jnp.float32),
        ],
    )(agg, cnt3, cnt3, x, Wl, bl2, Wr, Wcat)


def kernel(x, edge_attr, params, edge_index, batch, pat_idxs):
    f32 = jnp.float32
    x = x.at[:, :12].set(x[:, :12] / jnp.max(x[:, :12], axis=0, keepdims=True))
    b = batch
    src = edge_index[0]
    dst = edge_index[1]
    onehot = b[:, None] == jnp.arange(_G, dtype=b.dtype)[None, :]
    pad = _EPAD - _E
    srcp = jnp.concatenate([src, jnp.zeros((pad,), jnp.int32)])
    dstp = jnp.concatenate([dst, jnp.full((pad,), _N, jnp.int32)])
    node_valid = jnp.ones((_N,), bool)
    selv = jnp.concatenate([jnp.ones((_N,), f32),
                            jnp.zeros((_NPAD - _N,), f32)])
    xpad = jnp.concatenate([x, jnp.zeros((_NPAD - _N, _D), f32)])
    xs = []
    for i in range(3):
        agg, cnt = _mp_kernel(xpad, srcp, dstp, selv)
        bl2 = params['conv%d_bl' % i].reshape(1, _D)
        Wcat = jnp.concatenate(
            [params['pool%d_Wrel' % i], params['pool%d_Wroot' % i],
             jnp.zeros((_D, 6), f32)], axis=1)
        xn, pr = _dense(agg, cnt, xpad, params['conv%d_Wl' % i], bl2,
                        params['conv%d_Wr' % i], Wcat)
        sagg, _ = _mp_kernel(xn, srcp, dstp, selv)
        s = ((sagg[0, :_N] + sagg[1, :_N]) @ params['pool%d_Wrel' % i][:, 0]
             + params['pool%d_brel' % i][0] + pr[:_N, 1])
        bits = jax.lax.bitcast_convert_type(s, jnp.uint32)
        msk = jnp.where((bits >> 31).astype(jnp.int32) == 1,
                        jnp.uint32(0xFFFFFFFF), jnp.uint32(0x80000000))
        key = jnp.where(node_valid, bits ^ msk, jnp.uint32(0))
        vcount = jnp.sum(onehot & node_valid[:, None], axis=0).astype(f32)
        k = jnp.ceil(0.2 * vcount).astype(jnp.int32)

        def bs_body(_, lohi):
            lo, hi = lohi
            mid = lo + (hi - lo) // 2
            tn = jnp.sum(jnp.where(onehot, mid[None, :], jnp.uint32(0)),
                         axis=1)
            cnt = jnp.sum((onehot & (key > tn)[:, None]).astype(jnp.int32),
                          axis=0)
            ge = cnt >= k
            return jnp.where(ge, mid + 1, lo), jnp.where(ge, hi, mid)

        lo, hi = lax.fori_loop(0, 32, bs_body,
                               (jnp.zeros((_G,), jnp.uint32),
                                jnp.full((_G,), 0xFFFFFFFF, jnp.uint32)))
        Tn = jnp.sum(jnp.where(onehot, lo[None, :], jnp.uint32(0)), axis=1)
        selhard = node_valid & (key > Tn)
        tie = node_valid & (key == Tn)
        c1 = jnp.sum((onehot & selhard[:, None]).astype(jnp.int32), axis=0)
        m = k - c1
        tiecnt = jnp.sum((onehot & tie[:, None]).astype(jnp.int32), axis=0)
        base = jnp.concatenate([jnp.zeros((1,), jnp.int32),
                                jnp.cumsum(tiecnt)[:-1]])
        cs = jnp.cumsum(tie.astype(jnp.int32))
        basen = jnp.sum(jnp.where(onehot, base[None, :], 0), axis=1)
        mn = jnp.sum(jnp.where(onehot, m[None, :], 0), axis=1)
        sel = selhard | (tie & (cs - 1 - basen < mn))

        xcore = jnp.where(sel[:, None], xn[:_N] * jnp.tanh(s)[:, None], 0.0)
        xpad = jnp.concatenate([xcore, jnp.zeros((_NPAD - _N, _D), f32)])
        selv = jnp.concatenate([sel.astype(f32),
                                jnp.zeros((_NPAD - _N,), f32)])
        node_valid = sel
        oh_sel = onehot & sel[:, None]
        gmax = jnp.stack([
            jnp.max(jnp.where(oh_sel[:, g][:, None], xcore, -jnp.inf), axis=0)
            for g in range(_G)])
        gsum = jnp.dot(oh_sel.astype(f32).T, xcore,
                       preferred_element_type=f32)
        gcnt = jnp.sum(oh_sel, axis=0).astype(f32)
        gmean = gsum / jnp.maximum(gcnt, 1.0)[:, None]
        xs.append(jnp.concatenate([gmax, gmean], axis=1))
    h = xs[0] + xs[1] + xs[2]
    h = jax.nn.relu(h @ params['enc1_W'] + params['enc1_b'])
    h = jax.nn.relu(h @ params['enc2_W'] + params['enc2_b'])
    grade = jax.nn.log_softmax(h @ params['grade_W'] + params['grade_b'],
                               axis=1)
    hazard = jax.nn.sigmoid(h @ params['hazard_W'] + params['hazard_b']) * 6.0 - 3.0
    return (h, grade, hazard)
